# baseline (speedup 1.0000x reference)
import functools

import jax
import jax.numpy as jnp
from jax import lax
from jax.experimental import pallas as pl
from jax.experimental.pallas import tpu as pltpu
from jax.experimental.pallas import tpu_sc as plsc

_VOCAB = 1000000
_EMB = 64
_OUT = 32
_B = 4096
_L = 200
_NTOK = _B * _L

_MB = 4096
_NBLK = -(-_VOCAB // _MB)
_NGB = 64
_G = _NGB * _MB

_NC = 2
_NS = 16
_NW = _NC * _NS
_PER_W = _NTOK // _NW
_BPW = _B // _NW
_CB = 8
_CHUNK = _CB * _L
_NCHUNK = _BPW // _CB


def _proj_body(l0, l1, l2, l3, w_ref, b_ref, out_ref):
    for j, lref in enumerate((l0, l1, l2, l3)):
        prod = lax.dot_general(
            lref[...], w_ref[...], (((0,), (0,)), ((), ())),
            preferred_element_type=jnp.float32,
        )
        out_ref[:, 32 * j:32 * (j + 1)] = prod + b_ref[...]


def _lhs_spec(j):
    def index_map(i):
        return (0, jnp.minimum(j * _NGB + i, _NBLK - 1))

    return pl.BlockSpec((_EMB, _MB), index_map)


def _project_table(emb_table, fc_w, fc_b):
    embT = emb_table.T
    packed = pl.pallas_call(
        _proj_body,
        grid=(_NGB,),
        in_specs=[_lhs_spec(j) for j in range(4)] + [
            pl.BlockSpec((_EMB, _OUT), lambda i: (0, 0)),
            pl.BlockSpec((1, _OUT), lambda i: (0, 0)),
        ],
        out_specs=pl.BlockSpec((_MB, 128), lambda i: (i, 0)),
        out_shape=jax.ShapeDtypeStruct((_G, 128), jnp.float32),
    )(embT, embT, embT, embT, fc_w.T, fc_b.reshape(1, _OUT))
    return packed.reshape(4 * _G, _OUT)


_MESH = plsc.VectorSubcoreMesh(core_axis_name="c", subcore_axis_name="s")


def _make_gather(ntok, chunk):
    per_w = ntok // _NW
    nchunk = per_w // chunk

    @functools.partial(
        pl.kernel,
        mesh=_MESH,
        out_type=jax.ShapeDtypeStruct((ntok, _OUT), jnp.float32),
        scratch_types=[
            pltpu.VMEM((chunk,), jnp.int32),
            pltpu.VMEM((chunk, _OUT), jnp.float32),
            pltpu.SemaphoreType.DMA,
        ],
        compiler_params=pltpu.CompilerParams(use_tc_tiling_on_sc=False),
    )
    def gather_rows(proj_hbm, idx_hbm, out_hbm, idx_v, rows_v, sem):
        wid = lax.axis_index("s") * _NC + lax.axis_index("c")
        base = wid * per_w

        def body(j, carry):
            off = pl.multiple_of(base + j * chunk, 8)
            pltpu.sync_copy(idx_hbm.at[pl.ds(off, chunk)], idx_v)
            pltpu.async_copy(proj_hbm.at[idx_v], rows_v, sem).wait()
            pltpu.sync_copy(rows_v, out_hbm.at[pl.ds(off, chunk)])
            return carry

        lax.fori_loop(0, nchunk, body, 0)

    return gather_rows


_gather_all = _make_gather(_NTOK, _CHUNK)

_PER_B = _L * _OUT
_ROWS_B = _PER_B // 128


@functools.partial(
    pl.kernel,
    mesh=_MESH,
    out_type=jax.ShapeDtypeStruct((_B, _OUT, _L), jnp.float32),
    scratch_types=[
        pltpu.VMEM((_CB * _ROWS_B, 128), jnp.float32),
        pltpu.VMEM((1, _OUT, _L), jnp.float32),
        pltpu.SemaphoreType.DMA,
    ],
    compiler_params=pltpu.CompilerParams(
        use_tc_tiling_on_sc=True, needs_layout_passes=False
    ),
)
def _transpose_out(flat_hbm, out_hbm, src_v, dst_v, sem):
    wid = lax.axis_index("s") * _NC + lax.axis_index("c")
    b_base = wid * _BPW
    lam = lax.iota(jnp.int32, 16)

    def chunk_body(k, carry):
        b0 = b_base + k * _CB
        pltpu.sync_copy(flat_hbm.at[pl.ds(b0 * _ROWS_B, _CB * _ROWS_B)], src_v)

        def b_body(i, carry2):
            def o_body(o, carry3):
                s = i * _PER_B + o
                for kk in range(12):
                    e = (kk * 16 + lam) * _OUT + s
                    x = plsc.load_gather(
                        src_v, [lax.shift_right_logical(e, 7), e & 127]
                    )
                    dst_v[0, o, pl.ds(kk * 16, 16)] = x
                e = jnp.minimum(192 + lam, _L - 1) * _OUT + s
                x = plsc.load_gather(
                    src_v, [lax.shift_right_logical(e, 7), e & 127]
                )
                plsc.store_scatter(dst_v, [0 * lam, 0 * lam + o, 192 + lam], x, mask=lam < 8)
                return carry3

            lax.fori_loop(0, _OUT, o_body, 0)
            pltpu.sync_copy(dst_v, out_hbm.at[pl.ds(b0 + i, 1)])
            return carry2

        lax.fori_loop(0, _CB, b_body, 0)
        return carry

    lax.fori_loop(0, _BPW // _CB, chunk_body, 0)


def kernel(inputs_ids, input_lens, emb_table, fc_w, fc_b):
    del input_lens
    proj = _project_table(emb_table, fc_w, fc_b)
    v = inputs_ids.reshape(_NTOK).astype(jnp.int32)
    ids_flat = lax.shift_left(v & (_G - 1), 2) | lax.shift_right_logical(v, 18)
    flat = _gather_all(proj, ids_flat)
    outT = _transpose_out(flat.reshape(_NTOK * _OUT // 128, 128))
    return jnp.transpose(outT, (0, 2, 1))


# speedup vs baseline: 1.5247x; 1.5247x over previous
import functools

import jax
import jax.numpy as jnp
from jax import lax
from jax.experimental import pallas as pl
from jax.experimental.pallas import tpu as pltpu
from jax.experimental.pallas import tpu_sc as plsc

_VOCAB = 1000000
_EMB = 64
_OUT = 32
_B = 4096
_L = 200
_NTOK = _B * _L

_MB = 4096
_NBLK = -(-_VOCAB // _MB)
_NGB = 64
_G = _NGB * _MB

_NC = 2
_NS = 16
_NW = _NC * _NS
_PER_W = _NTOK // _NW
_BPW = _B // _NW
_CB = 8
_CHUNK = _CB * _L
_NCHUNK = _BPW // _CB


def _proj_body(l0, l1, l2, l3, w_ref, b_ref, out_ref):
    for j, lref in enumerate((l0, l1, l2, l3)):
        prod = lax.dot_general(
            lref[...], w_ref[...], (((0,), (0,)), ((), ())),
            preferred_element_type=jnp.float32,
        )
        out_ref[:, 32 * j:32 * (j + 1)] = prod + b_ref[...]


def _lhs_spec(j):
    def index_map(i):
        return (0, jnp.minimum(j * _NGB + i, _NBLK - 1))

    return pl.BlockSpec((_EMB, _MB), index_map)


def _project_table(emb_table, fc_w, fc_b):
    embT = emb_table.T
    packed = pl.pallas_call(
        _proj_body,
        grid=(_NGB,),
        in_specs=[_lhs_spec(j) for j in range(4)] + [
            pl.BlockSpec((_EMB, _OUT), lambda i: (0, 0)),
            pl.BlockSpec((1, _OUT), lambda i: (0, 0)),
        ],
        out_specs=pl.BlockSpec((_MB, 128), lambda i: (i, 0)),
        out_shape=jax.ShapeDtypeStruct((_G, 128), jnp.float32),
    )(embT, embT, embT, embT, fc_w.T, fc_b.reshape(1, _OUT))
    return packed.reshape(4 * _G, _OUT)


_MESH = plsc.VectorSubcoreMesh(core_axis_name="c", subcore_axis_name="s")


def _make_gather(ntok, chunk):
    per_w = ntok // _NW
    nchunk = per_w // chunk

    @functools.partial(
        pl.kernel,
        mesh=_MESH,
        out_type=jax.ShapeDtypeStruct((ntok, _OUT), jnp.float32),
        scratch_types=[
            pltpu.VMEM((chunk,), jnp.int32),
            pltpu.VMEM((chunk, _OUT), jnp.float32),
            pltpu.SemaphoreType.DMA,
        ],
        compiler_params=pltpu.CompilerParams(use_tc_tiling_on_sc=False),
    )
    def gather_rows(proj_hbm, idx_hbm, out_hbm, idx_v, rows_v, sem):
        wid = lax.axis_index("s") * _NC + lax.axis_index("c")
        base = wid * per_w

        def body(j, carry):
            off = pl.multiple_of(base + j * chunk, 8)
            pltpu.sync_copy(idx_hbm.at[pl.ds(off, chunk)], idx_v)
            pltpu.async_copy(proj_hbm.at[idx_v], rows_v, sem).wait()
            pltpu.sync_copy(rows_v, out_hbm.at[pl.ds(off, chunk)])
            return carry

        lax.fori_loop(0, nchunk, body, 0)

    return gather_rows


_gather_all = _make_gather(_NTOK, _CHUNK)


def kernel(inputs_ids, input_lens, emb_table, fc_w, fc_b):
    del input_lens
    proj = _project_table(emb_table, fc_w, fc_b)
    v = inputs_ids.reshape(_NTOK).astype(jnp.int32)
    ids_flat = lax.shift_left(v & (_G - 1), 2) | lax.shift_right_logical(v, 18)
    flat = _gather_all(proj, ids_flat)
    return flat.reshape(_B, _L, _OUT)


# speedup vs baseline: 1.5307x; 1.0039x over previous
import functools

import jax
import jax.numpy as jnp
from jax import lax
from jax.experimental import pallas as pl
from jax.experimental.pallas import tpu as pltpu
from jax.experimental.pallas import tpu_sc as plsc

_VOCAB = 1000000
_EMB = 64
_OUT = 32
_B = 4096
_L = 200
_NTOK = _B * _L

_MB = 8192
_NBLK = -(-_VOCAB // _MB)
_NGB = 32
_G = _NGB * _MB

_NC = 2
_NS = 16
_NW = _NC * _NS
_PER_W = _NTOK // _NW
_BPW = _B // _NW
_CB = 8
_CHUNK = _CB * _L
_NCHUNK = _BPW // _CB


def _proj_body(l0, l1, l2, l3, w_ref, b_ref, out_ref):
    for j, lref in enumerate((l0, l1, l2, l3)):
        prod = lax.dot_general(
            lref[...], w_ref[...], (((0,), (0,)), ((), ())),
            preferred_element_type=jnp.float32,
        )
        out_ref[:, 32 * j:32 * (j + 1)] = prod + b_ref[...]


def _lhs_spec(j):
    def index_map(i):
        return (0, jnp.minimum(j * _NGB + i, _NBLK - 1))

    return pl.BlockSpec((_EMB, _MB), index_map)


def _project_table(emb_table, fc_w, fc_b):
    embT = emb_table.T
    packed = pl.pallas_call(
        _proj_body,
        grid=(_NGB,),
        in_specs=[_lhs_spec(j) for j in range(4)] + [
            pl.BlockSpec((_EMB, _OUT), lambda i: (0, 0)),
            pl.BlockSpec((1, _OUT), lambda i: (0, 0)),
        ],
        out_specs=pl.BlockSpec((_MB, 128), lambda i: (i, 0)),
        out_shape=jax.ShapeDtypeStruct((_G, 128), jnp.float32),
        compiler_params=pltpu.CompilerParams(vmem_limit_bytes=50 * 2**20),
    )(embT, embT, embT, embT, fc_w.T, fc_b.reshape(1, _OUT))
    return packed.reshape(4 * _G, _OUT)


_MESH = plsc.VectorSubcoreMesh(core_axis_name="c", subcore_axis_name="s")


def _make_gather(ntok, chunk):
    per_w = ntok // _NW
    nchunk = per_w // chunk

    @functools.partial(
        pl.kernel,
        mesh=_MESH,
        out_type=jax.ShapeDtypeStruct((ntok, _OUT), jnp.float32),
        scratch_types=[
            pltpu.VMEM((chunk,), jnp.int32),
            pltpu.VMEM((chunk, _OUT), jnp.float32),
            pltpu.SemaphoreType.DMA,
        ],
        compiler_params=pltpu.CompilerParams(use_tc_tiling_on_sc=False),
    )
    def gather_rows(proj_hbm, idx_hbm, out_hbm, idx_v, rows_v, sem):
        wid = lax.axis_index("s") * _NC + lax.axis_index("c")
        base = wid * per_w

        def body(j, carry):
            off = pl.multiple_of(base + j * chunk, 8)
            pltpu.sync_copy(idx_hbm.at[pl.ds(off, chunk)], idx_v)
            pltpu.async_copy(proj_hbm.at[idx_v], rows_v, sem).wait()
            pltpu.sync_copy(rows_v, out_hbm.at[pl.ds(off, chunk)])
            return carry

        lax.fori_loop(0, nchunk, body, 0)

    return gather_rows


_gather_all = _make_gather(_NTOK, _CHUNK)


def kernel(inputs_ids, input_lens, emb_table, fc_w, fc_b):
    del input_lens
    proj = _project_table(emb_table, fc_w, fc_b)
    v = inputs_ids.reshape(_NTOK).astype(jnp.int32)
    ids_flat = lax.shift_left(v & (_G - 1), 2) | lax.shift_right_logical(v, 18)
    flat = _gather_all(proj, ids_flat)
    return flat.reshape(_B, _L, _OUT)


# speedup vs baseline: 1.5550x; 1.0159x over previous
import functools

import jax
import jax.numpy as jnp
from jax import lax
from jax.experimental import pallas as pl
from jax.experimental.pallas import tpu as pltpu
from jax.experimental.pallas import tpu_sc as plsc

_VOCAB = 1000000
_EMB = 64
_OUT = 32
_B = 4096
_L = 200
_NTOK = _B * _L

_MB = 8192
_NBLK = -(-_VOCAB // _MB)
_NGB = 32
_G = _NGB * _MB

_NC = 2
_NS = 16
_NW = _NC * _NS
_PER_W = _NTOK // _NW
_BPW = _B // _NW
_CB = 16
_CHUNK = _CB * _L
_NCHUNK = _BPW // _CB


def _proj_body(l0, l1, l2, l3, w_ref, b_ref, out_ref):
    for j, lref in enumerate((l0, l1, l2, l3)):
        prod = lax.dot_general(
            lref[...], w_ref[...], (((0,), (0,)), ((), ())),
            preferred_element_type=jnp.float32,
        )
        out_ref[:, 32 * j:32 * (j + 1)] = prod + b_ref[...]


def _lhs_spec(j):
    def index_map(i):
        return (0, jnp.minimum(j * _NGB + i, _NBLK - 1))

    return pl.BlockSpec((_EMB, _MB), index_map)


def _project_table(emb_table, fc_w, fc_b):
    embT = emb_table.T
    packed = pl.pallas_call(
        _proj_body,
        grid=(_NGB,),
        in_specs=[_lhs_spec(j) for j in range(4)] + [
            pl.BlockSpec((_EMB, _OUT), lambda i: (0, 0)),
            pl.BlockSpec((1, _OUT), lambda i: (0, 0)),
        ],
        out_specs=pl.BlockSpec((_MB, 128), lambda i: (i, 0)),
        out_shape=jax.ShapeDtypeStruct((_G, 128), jnp.float32),
        compiler_params=pltpu.CompilerParams(vmem_limit_bytes=50 * 2**20),
    )(embT, embT, embT, embT, fc_w.T, fc_b.reshape(1, _OUT))
    return packed.reshape(4 * _G, _OUT)


_MESH = plsc.VectorSubcoreMesh(core_axis_name="c", subcore_axis_name="s")


def _make_gather(ntok, chunk):
    per_w = ntok // _NW
    nchunk = per_w // chunk

    @functools.partial(
        pl.kernel,
        mesh=_MESH,
        out_type=jax.ShapeDtypeStruct((ntok, _OUT), jnp.float32),
        scratch_types=[
            pltpu.VMEM((chunk,), jnp.int32),
            pltpu.VMEM((chunk, _OUT), jnp.float32),
            pltpu.SemaphoreType.DMA,
        ],
        compiler_params=pltpu.CompilerParams(use_tc_tiling_on_sc=False),
    )
    def gather_rows(proj_hbm, idx_hbm, out_hbm, idx_v, rows_v, sem):
        wid = lax.axis_index("s") * _NC + lax.axis_index("c")
        base = wid * per_w

        def body(j, carry):
            off = pl.multiple_of(base + j * chunk, 8)
            pltpu.sync_copy(idx_hbm.at[pl.ds(off, chunk)], idx_v)
            pltpu.async_copy(proj_hbm.at[idx_v], rows_v, sem).wait()
            pltpu.sync_copy(rows_v, out_hbm.at[pl.ds(off, chunk)])
            return carry

        lax.fori_loop(0, nchunk, body, 0)

    return gather_rows


_gather_all = _make_gather(_NTOK, _CHUNK)


def kernel(inputs_ids, input_lens, emb_table, fc_w, fc_b):
    del input_lens
    proj = _project_table(emb_table, fc_w, fc_b)
    v = inputs_ids.reshape(_NTOK).astype(jnp.int32)
    ids_flat = lax.shift_left(v & (_G - 1), 2) | lax.shift_right_logical(v, 18)
    flat = _gather_all(proj, ids_flat)
    return flat.reshape(_B, _L, _OUT)
